# baseline (speedup 1.0000x reference)
import dataclasses
import functools

import jax
import jax.numpy as jnp
from jax import lax
from jax.experimental import pallas as pl
from jax.experimental.pallas import tpu as pltpu
from jax.experimental.pallas import tpu_sc as plsc

_B, _N, _M = 512, 128, 128
_TOTAL = _B * _N * _M
_NUM_CORR = 2000
_NW = 32
_PER_W = _TOTAL // _NW
_CHUNK = 16384
_NCH = _PER_W // _CHUNK
_LANES = 16
_UNROLL = 8
_SIGN = -(2**31)
_POSM = 2**31 - 1


def _shr(v, k):
    return lax.shift_right_logical(v, jnp.full((_LANES,), k, jnp.int32))


def _pb1(ub, idx, s0, s1):
    return None, _shr(ub, 20)


def _pb2(ub, idx, s0, s1):
    return _shr(ub, 20) == s0, _shr(ub, 8) & jnp.int32(0xFFF)


def _pb3(ub, idx, s0, s1):
    return _shr(ub, 8) == s0, ub & jnp.int32(0xFF)


def _pb4(ub, idx, s0, s1):
    return ub == s0, _shr(idx, 11)


def _pb5(ub, idx, s0, s1):
    return (ub == s0) & (_shr(idx, 11) == s1), idx & jnp.int32(0x7FF)


def _compiler_params():
    cp = pltpu.CompilerParams()
    if "needs_layout_passes" in pltpu.CompilerParams.__dataclass_fields__:
        cp = dataclasses.replace(cp, needs_layout_passes=False)
    return cp


@functools.cache
def _make_sc_pass(nbins, pred_bin_fn):
    mesh = plsc.VectorSubcoreMesh(core_axis_name="c", subcore_axis_name="s")

    @functools.partial(
        pl.kernel,
        out_type=jax.ShapeDtypeStruct((_NW, nbins * _LANES), jnp.int32),
        mesh=mesh,
        scratch_types=[
            pltpu.VMEM((_CHUNK,), jnp.float32),
            pltpu.VMEM((_CHUNK,), jnp.float32),
            pltpu.VMEM((32,), jnp.int32),
            pltpu.VMEM((nbins * _LANES,), jnp.int32),
            pltpu.SemaphoreType.DMA,
            pltpu.SemaphoreType.DMA,
            pltpu.SemaphoreType.DMA,
        ],
        compiler_params=_compiler_params(),
    )
    def kern(data_hbm, state_hbm, hist_hbm, buf0, buf1, state_v, hist,
             sem0, sem1, sems):
        wid = lax.axis_index("c") * 16 + lax.axis_index("s")
        base = wid * _PER_W
        pltpu.async_copy(state_hbm, state_v, sems).wait()
        s0 = state_v[pl.ds(0, _LANES)]
        s1 = state_v[pl.ds(_LANES, _LANES)]
        lanes = lax.iota(jnp.int32, _LANES)
        ones = jnp.ones((_LANES,), jnp.int32)
        zeros = jnp.zeros((_LANES,), jnp.int32)
        c31 = jnp.full((_LANES,), 31, jnp.int32)
        csign = jnp.full((_LANES,), _SIGN, jnp.int32)

        def start(chunk, bref, sem):
            pltpu.async_copy(
                data_hbm.at[pl.ds(base + chunk * _CHUNK, _CHUNK)], bref, sem
            )

        def drain(bref, sem):
            pltpu.make_async_copy(
                data_hbm.at[pl.ds(base, _CHUNK)], bref, sem
            ).wait()

        def process(bref, cbase):
            @pl.loop(0, _CHUNK, step=_LANES * _UNROLL)
            def _(i):
                bits = [
                    plsc.bitcast(bref[pl.ds(i + u * _LANES, _LANES)],
                                 jnp.int32)
                    for u in range(_UNROLL)
                ]
                ubs = [
                    b ^ (lax.shift_right_arithmetic(b, c31) | csign)
                    for b in bits
                ]
                pbs = [
                    pred_bin_fn(ubs[u], (cbase + i + u * _LANES) + lanes,
                                s0, s1)
                    for u in range(_UNROLL)
                ]
                addrs = [
                    (pbs[u][1] << 4) | ((lanes + u) & 15)
                    for u in range(_UNROLL)
                ]
                for u in range(_UNROLL):
                    pred = pbs[u][0]
                    if pred is None:
                        plsc.addupdate_scatter(hist, [addrs[u]], ones)
                    else:
                        plsc.addupdate_scatter(hist, [addrs[u]], ones,
                                               mask=pred)

        @pl.loop(0, nbins * _LANES, step=_LANES * _UNROLL)
        def _(i):
            for u in range(_UNROLL):
                hist[pl.ds(i + u * _LANES, _LANES)] = zeros

        start(0, buf0, sem0)

        @pl.loop(0, _NCH, step=2)
        def _(c):
            start(c + 1, buf1, sem1)
            drain(buf0, sem0)
            process(buf0, base + c * _CHUNK)

            @pl.when(c + 2 < _NCH)
            def _():
                start(c + 2, buf0, sem0)

            drain(buf1, sem1)
            process(buf1, base + (c + 1) * _CHUNK)

        pltpu.sync_copy(hist, hist_hbm.at[wid])

    return kern


def _sc_pass1(d, s):
    return _make_sc_pass(4096, _pb1)(d, s)


def _sc_pass2(d, s):
    return _make_sc_pass(4096, _pb2)(d, s)


def _sc_pass3(d, s):
    return _make_sc_pass(256, _pb3)(d, s)


def _sc_pass4(d, s):
    return _make_sc_pass(4096, _pb4)(d, s)


def _sc_pass5(d, s):
    return _make_sc_pass(2048, _pb5)(d, s)


def _desc_step(hsum, k):
    suffix = jnp.cumsum(hsum[::-1])[::-1]
    c = jnp.sum((suffix >= k).astype(jnp.int32)) - 1
    above = suffix[c] - hsum[c]
    return c, k - above


def _asc_step(hsum, k):
    csum = jnp.cumsum(hsum)
    c = jnp.sum((csum < k).astype(jnp.int32))
    below = csum[c] - hsum[c]
    return c, k - below


def _state(a, b):
    return jnp.concatenate(
        [jnp.full((_LANES,), a, jnp.int32), jnp.full((_LANES,), b, jnp.int32)]
    )


_BB = 8


def _mark_body(score_ref, refm_ref, srcm_ref, thr_ref, corr_ref, msk_ref):
    v = thr_ref[0]
    cut = thr_ref[1]
    g = pl.program_id(0)
    x = score_ref[...]
    bits = lax.bitcast_convert_type(x, jnp.int32)
    mkey = jnp.where(bits < 0, bits ^ _POSM, bits)
    bi = lax.broadcasted_iota(jnp.int32, (_BB, _N, _M), 0)
    ri = lax.broadcasted_iota(jnp.int32, (_BB, _N, _M), 1)
    ci = lax.broadcasted_iota(jnp.int32, (_BB, _N, _M), 2)
    flat = ((g * _BB + bi) * _N + ri) * _M + ci
    sel = (mkey > v) | ((mkey == v) & (flat <= cut))
    rm = refm_ref[...] > 0
    sm = srcm_ref[...] > 0
    corr = sel & rm & sm
    corr_ref[...] = corr
    msk_ref[...] = jnp.where(corr, x, jnp.float32(0.0))


def _mark(score_mat, refm, srcm, thr):
    grid = _B // _BB
    return pl.pallas_call(
        _mark_body,
        grid=(grid,),
        in_specs=[
            pl.BlockSpec((_BB, _N, _M), lambda g: (g, 0, 0)),
            pl.BlockSpec((_BB, _N, 1), lambda g: (g, 0, 0)),
            pl.BlockSpec((_BB, 1, _M), lambda g: (g, 0, 0)),
            pl.BlockSpec(memory_space=pltpu.SMEM),
        ],
        out_specs=[
            pl.BlockSpec((_BB, _N, _M), lambda g: (g, 0, 0)),
            pl.BlockSpec((_BB, _N, _M), lambda g: (g, 0, 0)),
        ],
        out_shape=[
            jax.ShapeDtypeStruct((_B, _N, _M), jnp.bool_),
            jax.ShapeDtypeStruct((_B, _N, _M), jnp.float32),
        ],
    )(score_mat, refm, srcm, thr)


def kernel(score_mat, ref_knn_masks, src_knn_masks):
    flat = score_mat.reshape(_TOTAL)
    k = jnp.int32(_NUM_CORR)

    def _red(h, nbins):
        return h.reshape(_NW, nbins, _LANES).sum(axis=(0, 2))

    h1 = _red(_sc_pass1(flat, _state(0, 0)), 4096)
    c1, k = _desc_step(h1, k)
    h2 = _red(_sc_pass2(flat, _state(c1, 0)), 4096)
    c2, k = _desc_step(h2, k)
    p12 = (c1 << 12) | c2
    h3 = _red(_sc_pass3(flat, _state(p12, 0)), 256)
    c3, k = _desc_step(h3, k)
    ukeyv = (p12 << 8) | c3
    n_eq = h3[c3]

    def _tiebreak(args):
        flat_, ukeyv_, k_ = args
        h4 = _red(_sc_pass4(flat_, _state(ukeyv_, 0)), 4096)
        c4, k4 = _asc_step(h4, k_)
        h5 = _red(_sc_pass5(flat_, _state(ukeyv_, c4)), 2048)
        c5, _ = _asc_step(h5, k4)
        return (c4 << 11) | c5

    def _all_equals(args):
        return jnp.int32(_TOTAL)

    cutoff = lax.cond(k < n_eq, _tiebreak, _all_equals, (flat, ukeyv, k))
    v_mkey = ukeyv ^ _SIGN

    thr = jnp.stack([v_mkey, cutoff]).astype(jnp.int32)
    refm = ref_knn_masks.astype(jnp.int32)[:, :, None]
    srcm = src_knn_masks.astype(jnp.int32)[:, None, :]
    return _mark(score_mat, refm, srcm, thr)


# speedup vs baseline: 1.1916x; 1.1916x over previous
import dataclasses
import functools

import jax
import jax.numpy as jnp
from jax import lax
from jax.experimental import pallas as pl
from jax.experimental.pallas import tpu as pltpu
from jax.experimental.pallas import tpu_sc as plsc

_B, _N, _M = 512, 128, 128
_TOTAL = _B * _N * _M
_NUM_CORR = 2000
_NW = 32
_PER_W = _TOTAL // _NW
_CHUNK = 16384
_NCH = _PER_W // _CHUNK
_LANES = 16
_UNROLL = 8
_SIGN = -(2**31)
_POSM = 2**31 - 1


def _shr(v, k):
    return lax.shift_right_logical(v, jnp.full((_LANES,), k, jnp.int32))


def _pb1(ub, idx, s0, s1):
    return None, _shr(ub, 20)


def _pb2(ub, idx, s0, s1):
    return _shr(ub, 20) == s0, _shr(ub, 8) & jnp.int32(0xFFF)


def _pb3(ub, idx, s0, s1):
    return _shr(ub, 8) == s0, ub & jnp.int32(0xFF)


def _pb4(ub, idx, s0, s1):
    return ub == s0, _shr(idx, 11)


def _pb5(ub, idx, s0, s1):
    return (ub == s0) & (_shr(idx, 11) == s1), idx & jnp.int32(0x7FF)


def _compiler_params():
    cp = pltpu.CompilerParams()
    if "needs_layout_passes" in pltpu.CompilerParams.__dataclass_fields__:
        cp = dataclasses.replace(cp, needs_layout_passes=False)
    return cp


_PERLANE = 64


@functools.cache
def _make_sc_pass(nbins, pred_bin_fn, compact=False):
    mesh = plsc.VectorSubcoreMesh(core_axis_name="c", subcore_axis_name="s")
    out_type = [jax.ShapeDtypeStruct((_NW, nbins * _LANES), jnp.int32)]
    scratch = [
        pltpu.VMEM((_CHUNK,), jnp.float32),
        pltpu.VMEM((_CHUNK,), jnp.float32),
        pltpu.VMEM((32,), jnp.int32),
        pltpu.VMEM((nbins * _LANES,), jnp.int32),
    ]
    if compact:
        out_type += [
            jax.ShapeDtypeStruct((_NW, _PERLANE * _LANES), jnp.int32),
            jax.ShapeDtypeStruct((_NW, _LANES), jnp.int32),
        ]
        scratch += [
            pltpu.VMEM((_PERLANE * _LANES,), jnp.int32),
            pltpu.VMEM((_LANES,), jnp.int32),
        ]
    scratch += [pltpu.SemaphoreType.DMA] * 3

    @functools.partial(
        pl.kernel,
        out_type=tuple(out_type) if compact else out_type[0],
        mesh=mesh,
        scratch_types=scratch,
        compiler_params=_compiler_params(),
    )
    def kern(*refs):
        if compact:
            (data_hbm, state_hbm, hist_hbm, comp_hbm, cnts_hbm,
             buf0, buf1, state_v, hist, comp_v, cnt_v,
             sem0, sem1, sems) = refs
        else:
            (data_hbm, state_hbm, hist_hbm, buf0, buf1, state_v, hist,
             sem0, sem1, sems) = refs
        wid = lax.axis_index("c") * 16 + lax.axis_index("s")
        base = wid * _PER_W
        pltpu.async_copy(state_hbm, state_v, sems).wait()
        s0 = state_v[pl.ds(0, _LANES)]
        s1 = state_v[pl.ds(_LANES, _LANES)]
        lanes = lax.iota(jnp.int32, _LANES)
        ones = jnp.ones((_LANES,), jnp.int32)
        zeros = jnp.zeros((_LANES,), jnp.int32)
        c31 = jnp.full((_LANES,), 31, jnp.int32)
        csign = jnp.full((_LANES,), _SIGN, jnp.int32)

        def start(chunk, bref, sem):
            pltpu.async_copy(
                data_hbm.at[pl.ds(base + chunk * _CHUNK, _CHUNK)], bref, sem
            )

        def drain(bref, sem):
            pltpu.make_async_copy(
                data_hbm.at[pl.ds(base, _CHUNK)], bref, sem
            ).wait()

        def process(bref, cbase):
            @pl.loop(0, _CHUNK, step=_LANES * _UNROLL)
            def _(i):
                bits = [
                    plsc.bitcast(bref[pl.ds(i + u * _LANES, _LANES)],
                                 jnp.int32)
                    for u in range(_UNROLL)
                ]
                ubs = [
                    b ^ (lax.shift_right_arithmetic(b, c31) | csign)
                    for b in bits
                ]
                pbs = [
                    pred_bin_fn(ubs[u], (cbase + i + u * _LANES) + lanes,
                                s0, s1)
                    for u in range(_UNROLL)
                ]
                addrs = [
                    (pbs[u][1] << 4) | ((lanes + u) & 15)
                    for u in range(_UNROLL)
                ]
                for u in range(_UNROLL):
                    pred = pbs[u][0]
                    if pred is None:
                        plsc.addupdate_scatter(hist, [addrs[u]], ones)
                    else:
                        plsc.addupdate_scatter(hist, [addrs[u]], ones,
                                               mask=pred)
                if compact:
                    cmax = jnp.full((_LANES,), _PERLANE - 1, jnp.int32)
                    cnt = cnt_v[...]
                    for u in range(_UNROLL):
                        pred, bin_ = pbs[u]
                        idx = (cbase + i + u * _LANES) + lanes
                        packed = (bin_ << 23) | idx
                        slot = jnp.minimum(cnt, cmax)
                        caddr = (slot << 4) | lanes
                        plsc.store_scatter(comp_v, [caddr], packed,
                                           mask=pred)
                        cnt = cnt + pred.astype(jnp.int32)
                    cnt_v[...] = cnt

        @pl.loop(0, nbins * _LANES, step=_LANES * _UNROLL)
        def _(i):
            for u in range(_UNROLL):
                hist[pl.ds(i + u * _LANES, _LANES)] = zeros

        if compact:
            neg1 = jnp.full((_LANES,), -1, jnp.int32)
            cnt_v[...] = zeros

            @pl.loop(0, _PERLANE * _LANES, step=_LANES)
            def _(i):
                comp_v[pl.ds(i, _LANES)] = neg1

        start(0, buf0, sem0)

        @pl.loop(0, _NCH, step=2)
        def _(c):
            start(c + 1, buf1, sem1)
            drain(buf0, sem0)
            process(buf0, base + c * _CHUNK)

            @pl.when(c + 2 < _NCH)
            def _():
                start(c + 2, buf0, sem0)

            drain(buf1, sem1)
            process(buf1, base + (c + 1) * _CHUNK)

        pltpu.sync_copy(hist, hist_hbm.at[wid])
        if compact:
            pltpu.sync_copy(comp_v, comp_hbm.at[wid])
            pltpu.sync_copy(cnt_v, cnts_hbm.at[wid])

    return kern


def _sc_pass1(d, s):
    return _make_sc_pass(4096, _pb1)(d, s)


def _sc_pass2(d, s):
    return _make_sc_pass(4096, _pb2)(d, s)


def _sc_pass3(d, s):
    return _make_sc_pass(256, _pb3, compact=True)(d, s)


def _sc_pass4(d, s):
    return _make_sc_pass(4096, _pb4)(d, s)


def _sc_pass5(d, s):
    return _make_sc_pass(2048, _pb5)(d, s)


def _desc_step(hsum, k):
    suffix = jnp.cumsum(hsum[::-1])[::-1]
    c = jnp.sum((suffix >= k).astype(jnp.int32)) - 1
    above = suffix[c] - hsum[c]
    return c, k - above


def _asc_step(hsum, k):
    csum = jnp.cumsum(hsum)
    c = jnp.sum((csum < k).astype(jnp.int32))
    below = csum[c] - hsum[c]
    return c, k - below


def _state(a, b):
    return jnp.concatenate(
        [jnp.full((_LANES,), a, jnp.int32), jnp.full((_LANES,), b, jnp.int32)]
    )


def _tb_body(comp_ref, st_ref, out_ref):
    c3 = st_ref[0]
    k4 = st_ref[1]
    comp = comp_ref[...]
    key8 = comp >> 23
    idxs = comp & jnp.int32(0x7FFFFF)
    big = jnp.where(key8 == c3, idxs, jnp.int32(2**23))

    def body(j, prefix):
        try_ = prefix | jnp.left_shift(jnp.int32(1), 22 - j)
        cnt = jnp.sum((big < try_).astype(jnp.int32))
        return jnp.where(cnt < k4, try_, prefix)

    out_ref[0] = lax.fori_loop(0, 23, body, jnp.int32(0))


def _tb_fast(comp, c3, k4):
    st = jnp.stack([c3, k4]).astype(jnp.int32)
    out = pl.pallas_call(
        _tb_body,
        in_specs=[
            pl.BlockSpec(memory_space=pltpu.VMEM),
            pl.BlockSpec(memory_space=pltpu.SMEM),
        ],
        out_specs=pl.BlockSpec(memory_space=pltpu.SMEM),
        out_shape=jax.ShapeDtypeStruct((1,), jnp.int32),
    )(comp, st)
    return out[0]


_BB = 8


def _mark_body(score_ref, refm_ref, srcm_ref, thr_ref, corr_ref, msk_ref):
    v = thr_ref[0]
    cut = thr_ref[1]
    g = pl.program_id(0)
    x = score_ref[...]
    bits = lax.bitcast_convert_type(x, jnp.int32)
    mkey = jnp.where(bits < 0, bits ^ _POSM, bits)
    bi = lax.broadcasted_iota(jnp.int32, (_BB, _N, _M), 0)
    ri = lax.broadcasted_iota(jnp.int32, (_BB, _N, _M), 1)
    ci = lax.broadcasted_iota(jnp.int32, (_BB, _N, _M), 2)
    flat = ((g * _BB + bi) * _N + ri) * _M + ci
    sel = (mkey > v) | ((mkey == v) & (flat <= cut))
    rm = refm_ref[...] > 0
    sm = srcm_ref[...] > 0
    corr = sel & rm & sm
    corr_ref[...] = corr
    msk_ref[...] = jnp.where(corr, x, jnp.float32(0.0))


def _mark(score_mat, refm, srcm, thr):
    grid = _B // _BB
    return pl.pallas_call(
        _mark_body,
        grid=(grid,),
        in_specs=[
            pl.BlockSpec((_BB, _N, _M), lambda g: (g, 0, 0)),
            pl.BlockSpec((_BB, _N, 1), lambda g: (g, 0, 0)),
            pl.BlockSpec((_BB, 1, _M), lambda g: (g, 0, 0)),
            pl.BlockSpec(memory_space=pltpu.SMEM),
        ],
        out_specs=[
            pl.BlockSpec((_BB, _N, _M), lambda g: (g, 0, 0)),
            pl.BlockSpec((_BB, _N, _M), lambda g: (g, 0, 0)),
        ],
        out_shape=[
            jax.ShapeDtypeStruct((_B, _N, _M), jnp.bool_),
            jax.ShapeDtypeStruct((_B, _N, _M), jnp.float32),
        ],
    )(score_mat, refm, srcm, thr)


def kernel(score_mat, ref_knn_masks, src_knn_masks):
    flat = score_mat.reshape(_TOTAL)
    k = jnp.int32(_NUM_CORR)

    def _red(h, nbins):
        return h.reshape(_NW, nbins, _LANES).sum(axis=(0, 2))

    h1 = _red(_sc_pass1(flat, _state(0, 0)), 4096)
    c1, k = _desc_step(h1, k)
    h2 = _red(_sc_pass2(flat, _state(c1, 0)), 4096)
    c2, k = _desc_step(h2, k)
    p12 = (c1 << 12) | c2
    h3_raw, comp, cnts = _sc_pass3(flat, _state(p12, 0))
    h3 = _red(h3_raw, 256)
    c3, k = _desc_step(h3, k)
    ukeyv = (p12 << 8) | c3

    def _tiebreak_full(args):
        flat_, ukeyv_, k_, _comp, _c3 = args
        h4 = _red(_sc_pass4(flat_, _state(ukeyv_, 0)), 4096)
        c4, k4 = _asc_step(h4, k_)
        h5 = _red(_sc_pass5(flat_, _state(ukeyv_, c4)), 2048)
        c5, _ = _asc_step(h5, k4)
        return (c4 << 11) | c5

    def _tiebreak_fast(args):
        _flat, _ukeyv, k_, comp_, c3_ = args
        return _tb_fast(comp_, c3_, k_)

    overflow = jnp.any(cnts > _PERLANE)
    cutoff = lax.cond(overflow, _tiebreak_full, _tiebreak_fast,
                      (flat, ukeyv, k, comp, c3))
    v_mkey = ukeyv ^ _SIGN

    thr = jnp.stack([v_mkey, cutoff]).astype(jnp.int32)
    refm = ref_knn_masks.astype(jnp.int32)[:, :, None]
    srcm = src_knn_masks.astype(jnp.int32)[:, None, :]
    return _mark(score_mat, refm, srcm, thr)


# speedup vs baseline: 1.1987x; 1.0060x over previous
import dataclasses
import functools

import jax
import jax.numpy as jnp
from jax import lax
from jax.experimental import pallas as pl
from jax.experimental.pallas import tpu as pltpu
from jax.experimental.pallas import tpu_sc as plsc

_B, _N, _M = 512, 128, 128
_TOTAL = _B * _N * _M
_NUM_CORR = 2000
_NW = 32
_PER_W = _TOTAL // _NW
_CHUNK = 16384
_NCH = _PER_W // _CHUNK
_LANES = 16
_UNROLL = 8
_SIGN = -(2**31)
_POSM = 2**31 - 1


def _shr(v, k):
    return lax.shift_right_logical(v, jnp.full((_LANES,), k, jnp.int32))


def _pb1(ub, idx, s0, s1):
    return None, _shr(ub, 20)


def _pb2(ub, idx, s0, s1):
    return _shr(ub, 20) == s0, _shr(ub, 8) & jnp.int32(0xFFF)


def _pb3(ub, idx, s0, s1):
    return _shr(ub, 8) == s0, ub & jnp.int32(0xFF)


def _pb4(ub, idx, s0, s1):
    return ub == s0, _shr(idx, 11)


def _pb5(ub, idx, s0, s1):
    return (ub == s0) & (_shr(idx, 11) == s1), idx & jnp.int32(0x7FF)


def _compiler_params():
    cp = pltpu.CompilerParams()
    if "needs_layout_passes" in pltpu.CompilerParams.__dataclass_fields__:
        cp = dataclasses.replace(cp, needs_layout_passes=False)
    return cp


_PERLANE = 64


@functools.cache
def _make_sc_pass(nbins, pred_bin_fn, compact=False):
    mesh = plsc.VectorSubcoreMesh(core_axis_name="c", subcore_axis_name="s")
    scratch = [
        pltpu.VMEM((_CHUNK,), jnp.float32),
        pltpu.VMEM((_CHUNK,), jnp.float32),
        pltpu.VMEM((32,), jnp.int32),
    ]
    if compact:
        out_type = (
            jax.ShapeDtypeStruct((_NW, _PERLANE * _LANES), jnp.int32),
            jax.ShapeDtypeStruct((_NW, _LANES), jnp.int32),
        )
        scratch += [
            pltpu.VMEM((_PERLANE * _LANES,), jnp.int32),
            pltpu.VMEM((_LANES,), jnp.int32),
        ]
    else:
        out_type = jax.ShapeDtypeStruct((_NW, nbins * _LANES), jnp.int32)
        scratch += [pltpu.VMEM((nbins * _LANES,), jnp.int32)]
    scratch += [pltpu.SemaphoreType.DMA] * 3

    @functools.partial(
        pl.kernel,
        out_type=out_type,
        mesh=mesh,
        scratch_types=scratch,
        compiler_params=_compiler_params(),
    )
    def kern(*refs):
        if compact:
            (data_hbm, state_hbm, comp_hbm, cnts_hbm,
             buf0, buf1, state_v, comp_v, cnt_v,
             sem0, sem1, sems) = refs
            hist = None
        else:
            (data_hbm, state_hbm, hist_hbm, buf0, buf1, state_v, hist,
             sem0, sem1, sems) = refs
        wid = lax.axis_index("c") * 16 + lax.axis_index("s")
        base = wid * _PER_W
        pltpu.async_copy(state_hbm, state_v, sems).wait()
        s0 = state_v[pl.ds(0, _LANES)]
        s1 = state_v[pl.ds(_LANES, _LANES)]
        lanes = lax.iota(jnp.int32, _LANES)
        ones = jnp.ones((_LANES,), jnp.int32)
        zeros = jnp.zeros((_LANES,), jnp.int32)
        c31 = jnp.full((_LANES,), 31, jnp.int32)
        csign = jnp.full((_LANES,), _SIGN, jnp.int32)

        def start(chunk, bref, sem):
            pltpu.async_copy(
                data_hbm.at[pl.ds(base + chunk * _CHUNK, _CHUNK)], bref, sem
            )

        def drain(bref, sem):
            pltpu.make_async_copy(
                data_hbm.at[pl.ds(base, _CHUNK)], bref, sem
            ).wait()

        def process(bref, cbase):
            @pl.loop(0, _CHUNK, step=_LANES * _UNROLL)
            def _(i):
                bits = [
                    plsc.bitcast(bref[pl.ds(i + u * _LANES, _LANES)],
                                 jnp.int32)
                    for u in range(_UNROLL)
                ]
                ubs = [
                    b ^ (lax.shift_right_arithmetic(b, c31) | csign)
                    for b in bits
                ]
                pbs = [
                    pred_bin_fn(ubs[u], (cbase + i + u * _LANES) + lanes,
                                s0, s1)
                    for u in range(_UNROLL)
                ]
                if not compact:
                    addrs = [
                        (pbs[u][1] << 4) | ((lanes + u) & 15)
                        for u in range(_UNROLL)
                    ]
                    for u in range(_UNROLL):
                        pred = pbs[u][0]
                        if pred is None:
                            plsc.addupdate_scatter(hist, [addrs[u]], ones)
                        else:
                            plsc.addupdate_scatter(hist, [addrs[u]], ones,
                                                   mask=pred)
                else:
                    cmax = jnp.full((_LANES,), _PERLANE - 1, jnp.int32)
                    cnt = cnt_v[...]
                    for u in range(_UNROLL):
                        pred, bin_ = pbs[u]
                        idx = (cbase + i + u * _LANES) + lanes
                        packed = (bin_ << 23) | idx
                        slot = jnp.minimum(cnt, cmax)
                        caddr = (slot << 4) | lanes
                        plsc.store_scatter(comp_v, [caddr], packed,
                                           mask=pred)
                        cnt = cnt + pred.astype(jnp.int32)
                    cnt_v[...] = cnt

        if compact:
            neg1 = jnp.full((_LANES,), -1, jnp.int32)
            cnt_v[...] = zeros

            @pl.loop(0, _PERLANE * _LANES, step=_LANES)
            def _(i):
                comp_v[pl.ds(i, _LANES)] = neg1
        else:
            @pl.loop(0, nbins * _LANES, step=_LANES * _UNROLL)
            def _(i):
                for u in range(_UNROLL):
                    hist[pl.ds(i + u * _LANES, _LANES)] = zeros

        start(0, buf0, sem0)

        @pl.loop(0, _NCH, step=2)
        def _(c):
            start(c + 1, buf1, sem1)
            drain(buf0, sem0)
            process(buf0, base + c * _CHUNK)

            @pl.when(c + 2 < _NCH)
            def _():
                start(c + 2, buf0, sem0)

            drain(buf1, sem1)
            process(buf1, base + (c + 1) * _CHUNK)

        if compact:
            pltpu.sync_copy(comp_v, comp_hbm.at[wid])
            pltpu.sync_copy(cnt_v, cnts_hbm.at[wid])
        else:
            pltpu.sync_copy(hist, hist_hbm.at[wid])

    return kern


def _sc_pass1(d, s):
    return _make_sc_pass(4096, _pb1)(d, s)


def _sc_pass2(d, s):
    return _make_sc_pass(4096, _pb2)(d, s)


def _sc_pass3(d, s):
    return _make_sc_pass(256, _pb3, compact=True)(d, s)


def _sc_pass4(d, s):
    return _make_sc_pass(4096, _pb4)(d, s)


def _sc_pass5(d, s):
    return _make_sc_pass(2048, _pb5)(d, s)


def _desc_step(hsum, k):
    suffix = jnp.cumsum(hsum[::-1])[::-1]
    c = jnp.sum((suffix >= k).astype(jnp.int32)) - 1
    above = suffix[c] - hsum[c]
    return c, k - above


def _asc_step(hsum, k):
    csum = jnp.cumsum(hsum)
    c = jnp.sum((csum < k).astype(jnp.int32))
    below = csum[c] - hsum[c]
    return c, k - below


def _state(a, b):
    return jnp.concatenate(
        [jnp.full((_LANES,), a, jnp.int32), jnp.full((_LANES,), b, jnp.int32)]
    )


def _tb_body(comp_ref, st_ref, out_ref):
    c3 = st_ref[0]
    k4 = st_ref[1]
    comp = comp_ref[...]
    key8 = comp >> 23
    idxs = comp & jnp.int32(0x7FFFFF)
    big = jnp.where(key8 == c3, idxs, jnp.int32(2**23))

    def body(j, prefix):
        try_ = prefix | jnp.left_shift(jnp.int32(1), 22 - j)
        cnt = jnp.sum((big < try_).astype(jnp.int32))
        return jnp.where(cnt < k4, try_, prefix)

    out_ref[0] = lax.fori_loop(0, 23, body, jnp.int32(0))


def _tb_fast(comp, c3, k4):
    st = jnp.stack([c3, k4]).astype(jnp.int32)
    out = pl.pallas_call(
        _tb_body,
        in_specs=[
            pl.BlockSpec(memory_space=pltpu.VMEM),
            pl.BlockSpec(memory_space=pltpu.SMEM),
        ],
        out_specs=pl.BlockSpec(memory_space=pltpu.SMEM),
        out_shape=jax.ShapeDtypeStruct((1,), jnp.int32),
    )(comp, st)
    return out[0]


_BB = 8


def _mark_body(score_ref, refm_ref, srcm_ref, thr_ref, corr_ref, msk_ref):
    v = thr_ref[0]
    cut = thr_ref[1]
    g = pl.program_id(0)
    x = score_ref[...]
    bits = lax.bitcast_convert_type(x, jnp.int32)
    mkey = jnp.where(bits < 0, bits ^ _POSM, bits)
    bi = lax.broadcasted_iota(jnp.int32, (_BB, _N, _M), 0)
    ri = lax.broadcasted_iota(jnp.int32, (_BB, _N, _M), 1)
    ci = lax.broadcasted_iota(jnp.int32, (_BB, _N, _M), 2)
    flat = ((g * _BB + bi) * _N + ri) * _M + ci
    sel = (mkey > v) | ((mkey == v) & (flat <= cut))
    rm = refm_ref[...] > 0
    sm = srcm_ref[...] > 0
    corr = sel & rm & sm
    corr_ref[...] = corr
    msk_ref[...] = jnp.where(corr, x, jnp.float32(0.0))


def _mark(score_mat, refm, srcm, thr):
    grid = _B // _BB
    return pl.pallas_call(
        _mark_body,
        grid=(grid,),
        in_specs=[
            pl.BlockSpec((_BB, _N, _M), lambda g: (g, 0, 0)),
            pl.BlockSpec((_BB, _N, 1), lambda g: (g, 0, 0)),
            pl.BlockSpec((_BB, 1, _M), lambda g: (g, 0, 0)),
            pl.BlockSpec(memory_space=pltpu.SMEM),
        ],
        out_specs=[
            pl.BlockSpec((_BB, _N, _M), lambda g: (g, 0, 0)),
            pl.BlockSpec((_BB, _N, _M), lambda g: (g, 0, 0)),
        ],
        out_shape=[
            jax.ShapeDtypeStruct((_B, _N, _M), jnp.bool_),
            jax.ShapeDtypeStruct((_B, _N, _M), jnp.float32),
        ],
    )(score_mat, refm, srcm, thr)


def kernel(score_mat, ref_knn_masks, src_knn_masks):
    flat = score_mat.reshape(_TOTAL)
    k = jnp.int32(_NUM_CORR)

    def _red(h, nbins):
        return h.reshape(_NW, nbins, _LANES).sum(axis=(0, 2))

    h1 = _red(_sc_pass1(flat, _state(0, 0)), 4096)
    c1, k = _desc_step(h1, k)
    h2 = _red(_sc_pass2(flat, _state(c1, 0)), 4096)
    c2, k = _desc_step(h2, k)
    p12 = (c1 << 12) | c2
    comp, cnts = _sc_pass3(flat, _state(p12, 0))

    def _finish_fast(args):
        _flat, p12_, k_, comp_ = args
        key8 = comp_ >> 23
        valid = comp_ != jnp.int32(-1)
        h3 = jnp.sum(
            (key8[:, :, None] == jnp.arange(256, dtype=jnp.int32))
            & valid[:, :, None],
            axis=(0, 1),
        ).astype(jnp.int32)
        c3, k3 = _desc_step(h3, k_)
        return (p12_ << 8) | c3, _tb_fast(comp_, c3, k3)

    def _finish_full(args):
        flat_, p12_, k_, _comp = args
        h3 = _red(_make_sc_pass(256, _pb3)(flat_, _state(p12_, 0)), 256)
        c3, k3 = _desc_step(h3, k_)
        ukeyv_ = (p12_ << 8) | c3
        h4 = _red(_sc_pass4(flat_, _state(ukeyv_, 0)), 4096)
        c4, k4 = _asc_step(h4, k3)
        h5 = _red(_sc_pass5(flat_, _state(ukeyv_, c4)), 2048)
        c5, _ = _asc_step(h5, k4)
        return ukeyv_, (c4 << 11) | c5

    overflow = jnp.any(cnts > _PERLANE)
    ukeyv, cutoff = lax.cond(overflow, _finish_full, _finish_fast,
                             (flat, p12, k, comp))
    v_mkey = ukeyv ^ _SIGN

    thr = jnp.stack([v_mkey, cutoff]).astype(jnp.int32)
    refm = ref_knn_masks.astype(jnp.int32)[:, :, None]
    srcm = src_knn_masks.astype(jnp.int32)[:, None, :]
    return _mark(score_mat, refm, srcm, thr)
